# baseline (speedup 1.0000x reference)
import jax
import jax.numpy as jnp
from jax.experimental import pallas as pl
from jax.experimental.pallas import tpu as pltpu

EPS = 0.01
TH = 512


def _h3max(v):
    n, w = v.shape
    zc = jnp.zeros((n, 1), v.dtype)
    left = jnp.concatenate([zc, v[:, :-1]], axis=1)
    right = jnp.concatenate([v[:, 1:], zc], axis=1)
    return jnp.maximum(jnp.maximum(left, right), v)


def _head_kernel(x_ref, ab_ref, be_ref, o_ref):
    t = pl.program_id(1)
    nt = pl.num_programs(1)
    x = x_ref[0]
    xp = jnp.maximum(x - EPS, 0.0)
    ab = jnp.where(t == 0, 0.0, jnp.maximum(ab_ref[0, 7:8] - EPS, 0.0))
    be = jnp.where(t == nt - 1, 0.0, jnp.maximum(be_ref[0, 0:1] - EPS, 0.0))

    th, w = xp.shape
    h3 = _h3max(xp)
    a3 = _h3max(ab)
    b3 = _h3max(be)

    up = jnp.concatenate([a3, h3[:th - 1]], axis=0)
    dn = jnp.concatenate([h3[1:], b3], axis=0)
    m3 = jnp.maximum(jnp.maximum(up, dn), h3)
    o_ref[0] = jnp.where(x > m3, xp, 0.0)


def kernel(x):
    B, H, W = x.shape
    T = H // TH
    tb = TH // 8
    return pl.pallas_call(
        _head_kernel,
        grid=(B, T),
        in_specs=[
            pl.BlockSpec((1, TH, W), lambda b, t: (b, t, 0)),
            pl.BlockSpec((1, 8, W),
                         lambda b, t: (b, jnp.maximum(t * tb - 1, 0), 0)),
            pl.BlockSpec((1, 8, W),
                         lambda b, t: (b, jnp.minimum((t + 1) * tb, H // 8 - 1), 0)),
        ],
        out_specs=pl.BlockSpec((1, TH, W), lambda b, t: (b, t, 0)),
        out_shape=jax.ShapeDtypeStruct((B, H, W), x.dtype),
        compiler_params=pltpu.CompilerParams(
            dimension_semantics=("parallel", "arbitrary")),
    )(x, x, x)


# speedup vs baseline: 1.0865x; 1.0865x over previous
import jax
import jax.numpy as jnp
from jax.experimental import pallas as pl
from jax.experimental.pallas import tpu as pltpu

EPS = 0.01
TH = 1024


def _h3max(v):
    n, w = v.shape
    zc = jnp.zeros((n, 1), v.dtype)
    left = jnp.concatenate([zc, v[:, :-1]], axis=1)
    right = jnp.concatenate([v[:, 1:], zc], axis=1)
    return jnp.maximum(jnp.maximum(left, right), v)


def _head_kernel(x_ref, ab_ref, be_ref, o_ref, h3_ref):
    t = pl.program_id(1)
    nt = pl.num_programs(1)
    ab = jnp.where(t == 0, 0.0, jnp.maximum(ab_ref[0, 7:8] - EPS, 0.0))
    be = jnp.where(t == nt - 1, 0.0, jnp.maximum(be_ref[0, 0:1] - EPS, 0.0))
    h3_ref[7:8] = _h3max(ab)
    h3_ref[TH + 8:TH + 9] = _h3max(be)
    for c in range(TH // 8):
        xc = x_ref[0, c * 8:(c + 1) * 8, :]
        h3_ref[c * 8 + 8:c * 8 + 16] = _h3max(jnp.maximum(xc - EPS, 0.0))
    for c in range(TH // 8):
        base = c * 8 + 8
        mid = h3_ref[base:base + 8]
        prev = h3_ref[base - 1:base]
        nxt = h3_ref[base + 8:base + 9]
        up = jnp.concatenate([prev, mid[:7]], axis=0)
        dn = jnp.concatenate([mid[1:], nxt], axis=0)
        m3 = jnp.maximum(jnp.maximum(up, dn), mid)
        xc = x_ref[0, c * 8:(c + 1) * 8, :]
        xpc = jnp.maximum(xc - EPS, 0.0)
        o_ref[0, c * 8:(c + 1) * 8, :] = jnp.where(xc > m3, xpc, 0.0)


def kernel(x):
    B, H, W = x.shape
    T = H // TH
    tb = TH // 8
    return pl.pallas_call(
        _head_kernel,
        grid=(B, T),
        in_specs=[
            pl.BlockSpec((1, TH, W), lambda b, t: (b, t, 0)),
            pl.BlockSpec((1, 8, W),
                         lambda b, t: (b, jnp.maximum(t * tb - 1, 0), 0)),
            pl.BlockSpec((1, 8, W),
                         lambda b, t: (b, jnp.minimum((t + 1) * tb, H // 8 - 1), 0)),
        ],
        out_specs=pl.BlockSpec((1, TH, W), lambda b, t: (b, t, 0)),
        out_shape=jax.ShapeDtypeStruct((B, H, W), x.dtype),
        scratch_shapes=[pltpu.VMEM((TH + 16, W), jnp.float32)],
        compiler_params=pltpu.CompilerParams(
            dimension_semantics=("parallel", "arbitrary")),
    )(x, x, x)


# speedup vs baseline: 1.1328x; 1.0426x over previous
import jax
import jax.numpy as jnp
from jax.experimental import pallas as pl
from jax.experimental.pallas import tpu as pltpu

EPS = 0.01
TH = 1024


def _h3max(v):
    n, w = v.shape
    zc = jnp.zeros((n, 1), v.dtype)
    left = jnp.concatenate([zc, v[:, :-1]], axis=1)
    right = jnp.concatenate([v[:, 1:], zc], axis=1)
    return jnp.maximum(jnp.maximum(left, right), v)


def _head_kernel(x_ref, ab_ref, be_ref, o_ref, h3_ref):
    t = pl.program_id(1)
    nt = pl.num_programs(1)
    ab = jnp.where(t == 0, 0.0, jnp.maximum(ab_ref[0, 7:8] - EPS, 0.0))
    be = jnp.where(t == nt - 1, 0.0, jnp.maximum(be_ref[0, 0:1] - EPS, 0.0))
    h3_ref[7:8] = _h3max(ab)
    h3_ref[TH + 8:TH + 9] = _h3max(be)
    for c in range(TH // 8):
        xc = x_ref[0, c * 8:(c + 1) * 8, :]
        xpc = jnp.maximum(xc - EPS, 0.0)
        o_ref[0, c * 8:(c + 1) * 8, :] = xpc
        h3_ref[c * 8 + 8:c * 8 + 16] = _h3max(xpc)
    for c in range(TH // 8):
        base = c * 8 + 8
        mid = h3_ref[base:base + 8]
        prev = h3_ref[base - 1:base]
        nxt = h3_ref[base + 8:base + 9]
        up = jnp.concatenate([prev, mid[:7]], axis=0)
        dn = jnp.concatenate([mid[1:], nxt], axis=0)
        m3 = jnp.maximum(jnp.maximum(up, dn), mid)
        xpc = o_ref[0, c * 8:(c + 1) * 8, :]
        o_ref[0, c * 8:(c + 1) * 8, :] = jnp.where(xpc > m3 - EPS, xpc, 0.0)


def kernel(x):
    B, H, W = x.shape
    T = H // TH
    tb = TH // 8
    return pl.pallas_call(
        _head_kernel,
        grid=(B, T),
        in_specs=[
            pl.BlockSpec((1, TH, W), lambda b, t: (b, t, 0)),
            pl.BlockSpec((1, 8, W),
                         lambda b, t: (b, jnp.maximum(t * tb - 1, 0), 0)),
            pl.BlockSpec((1, 8, W),
                         lambda b, t: (b, jnp.minimum((t + 1) * tb, H // 8 - 1), 0)),
        ],
        out_specs=pl.BlockSpec((1, TH, W), lambda b, t: (b, t, 0)),
        out_shape=jax.ShapeDtypeStruct((B, H, W), x.dtype),
        scratch_shapes=[pltpu.VMEM((TH + 16, W), jnp.float32)],
        compiler_params=pltpu.CompilerParams(
            dimension_semantics=("parallel", "arbitrary")),
    )(x, x, x)


# speedup vs baseline: 1.1834x; 1.0447x over previous
import jax
import jax.numpy as jnp
from jax.experimental import pallas as pl
from jax.experimental.pallas import tpu as pltpu

EPS = 0.01
TH = 1024


def _h3max(v):
    n, w = v.shape
    zc = jnp.zeros((n, 1), v.dtype)
    left = jnp.concatenate([zc, v[:, :-1]], axis=1)
    right = jnp.concatenate([v[:, 1:], zc], axis=1)
    return jnp.maximum(jnp.maximum(left, right), v)


def _head_kernel(x_ref, ab_ref, be_ref, o_ref, h3_ref):
    t = pl.program_id(1)
    nt = pl.num_programs(1)
    ab = jnp.where(t == 0, 0.0, jnp.maximum(ab_ref[0, 7:8] - EPS, 0.0))
    be = jnp.where(t == nt - 1, 0.0, jnp.maximum(be_ref[0, 0:1] - EPS, 0.0))
    h3_ref[7:8] = _h3max(ab)
    h3_ref[TH + 8:TH + 9] = _h3max(be)
    nck = TH // 8
    h3_prev = xp_prev = None
    for c in range(nck + 1):
        if c < nck:
            xc = x_ref[0, c * 8:(c + 1) * 8, :]
            xp_c = jnp.maximum(xc - EPS, 0.0)
            h3_c = _h3max(xp_c)
            h3_ref[c * 8 + 8:c * 8 + 16] = h3_c
        if c >= 1:
            k = c - 1
            base = k * 8 + 8
            prev = h3_ref[base - 1:base]
            nxt = h3_c[0:1] if c < nck else h3_ref[TH + 8:TH + 9]
            up = jnp.concatenate([prev, h3_prev[:7]], axis=0)
            dn = jnp.concatenate([h3_prev[1:], nxt], axis=0)
            m3 = jnp.maximum(jnp.maximum(up, dn), h3_prev)
            o_ref[0, k * 8:(k + 1) * 8, :] = jnp.where(
                xp_prev > m3 - EPS, xp_prev, 0.0)
        h3_prev, xp_prev = (h3_c, xp_c) if c < nck else (None, None)


def kernel(x):
    B, H, W = x.shape
    T = H // TH
    tb = TH // 8
    return pl.pallas_call(
        _head_kernel,
        grid=(B, T),
        in_specs=[
            pl.BlockSpec((1, TH, W), lambda b, t: (b, t, 0)),
            pl.BlockSpec((1, 8, W),
                         lambda b, t: (b, jnp.maximum(t * tb - 1, 0), 0)),
            pl.BlockSpec((1, 8, W),
                         lambda b, t: (b, jnp.minimum((t + 1) * tb, H // 8 - 1), 0)),
        ],
        out_specs=pl.BlockSpec((1, TH, W), lambda b, t: (b, t, 0)),
        out_shape=jax.ShapeDtypeStruct((B, H, W), x.dtype),
        scratch_shapes=[pltpu.VMEM((TH + 16, W), jnp.float32)],
        compiler_params=pltpu.CompilerParams(
            dimension_semantics=("parallel", "arbitrary")),
    )(x, x, x)
